# baseline (speedup 1.0000x reference)
import functools

import numpy as np

import jax
import jax.numpy as jnp
from jax import lax
from jax.experimental import pallas as pl
from jax.experimental.pallas import tpu as pltpu
from jax.experimental.pallas import tpu_sc as plsc

NC = 2
NS = 16
NW = NC * NS
D = 64
TB = 128


@functools.lru_cache(maxsize=None)
def _build(n_tok: int, seq: int):
    tiles_per_w = n_tok // (NW * TB)
    n_blocks = seq * tiles_per_w
    assert n_blocks % 2 == 0 and tiles_per_w * NW * TB == n_tok
    mesh = plsc.VectorSubcoreMesh(
        core_axis_name="c", subcore_axis_name="s",
        num_cores=NC, num_subcores=NS,
    )

    @functools.partial(
        pl.kernel,
        out_type=jax.ShapeDtypeStruct((seq, D // 8, n_tok // TB, 8, TB),
                                      jnp.float32),
        mesh=mesh,
        compiler_params=pltpu.CompilerParams(
            use_tc_tiling_on_sc=False, needs_layout_passes=False,
            disable_bounds_checks=True),
        scratch_types=[
            pltpu.VMEM((seq, tiles_per_w * TB), jnp.int32),
            pltpu.VMEM((TB, D), jnp.float32),
            pltpu.VMEM((TB, D), jnp.float32),
            pltpu.VMEM((D // 8, 8, TB), jnp.float32),
            pltpu.VMEM((D // 8, 8, TB), jnp.float32),
            pltpu.SemaphoreType.DMA,
            pltpu.SemaphoreType.DMA,
            pltpu.SemaphoreType.DMA,
            pltpu.SemaphoreType.DMA,
        ],
    )
    def body(idx_hbm, table_hbm, out_hbm, idx_v, rows_a, rows_b, t_a, t_b,
             gsem_a, gsem_b, osem_a, osem_b):
        wid = lax.axis_index("s") * NC + lax.axis_index("c")
        b0 = pl.multiple_of(wid * (tiles_per_w * TB), 8)
        lanes = lax.broadcasted_iota(jnp.int32, (16,), 0)
        rot = [(lanes + k) & 15 for k in range(16)]
        rot_hi = [lax.shift_right_logical(r, 3) for r in rot]
        rot_lo = [r & 7 for r in rot]

        pltpu.sync_copy(idx_hbm.at[:, pl.ds(b0, tiles_per_w * TB)], idx_v)

        def fire_gather(n, buf, sem):
            j = n // tiles_per_w
            tbl = n % tiles_per_w
            pltpu.async_copy(
                table_hbm.at[idx_v.at[j, pl.ds(tbl * TB, TB)]], buf, sem)

        def wait_gather(buf, sem):
            pltpu.make_async_copy(table_hbm.at[pl.ds(0, TB)], buf, sem).wait()

        def transpose(buf, t_v):
            @pl.loop(0, TB // 16, unroll=4)
            def _(bi):
                bmv = bi * 16 + lanes
                for cb in range(D // 16):
                    vals = [plsc.load_gather(buf, [bmv, cb * 16 + rot[k]])
                            for k in range(16)]
                    for k in range(16):
                        plsc.store_scatter(
                            t_v, [cb * 2 + rot_hi[k], rot_lo[k], bmv],
                            vals[k])

        def fire_out(n, t_v, sem):
            j = n // tiles_per_w
            tb = wid * tiles_per_w + (n % tiles_per_w)
            pltpu.async_copy(t_v, out_hbm.at[j, :, tb], sem)

        def wait_out(t_v, sem):
            pltpu.make_async_copy(t_v, out_hbm.at[0, :, 0], sem).wait()

        fire_gather(0, rows_a, gsem_a)
        fire_out(0, t_a, osem_a)
        fire_out(1, t_b, osem_b)

        @pl.loop(0, n_blocks, step=2)
        def _(g0):
            fire_gather(g0 + 1, rows_b, gsem_b)
            wait_gather(rows_a, gsem_a)
            wait_out(t_a, osem_a)
            transpose(rows_a, t_a)
            fire_out(g0, t_a, osem_a)

            @pl.when(g0 + 2 < n_blocks)
            def _():
                fire_gather(g0 + 2, rows_a, gsem_a)

            wait_gather(rows_b, gsem_b)
            wait_out(t_b, osem_b)
            transpose(rows_b, t_b)
            fire_out(g0 + 1, t_b, osem_b)

        wait_out(t_a, osem_a)
        wait_out(t_b, osem_b)

    return body


def kernel(token_ids, weights):
    n_tok, seq = token_ids.shape
    out5 = _build(n_tok, seq)(token_ids.T.astype(jnp.int32), weights)
    return out5.transpose(2, 4, 0, 1, 3).reshape(n_tok, seq, D)


# speedup vs baseline: 1.1256x; 1.1256x over previous
import functools

import numpy as np

import jax
import jax.numpy as jnp
from jax import lax
from jax.experimental import pallas as pl
from jax.experimental.pallas import tpu as pltpu
from jax.experimental.pallas import tpu_sc as plsc

NC = 2
NS = 16
NW = NC * NS
D = 64
TB = 128


@functools.lru_cache(maxsize=None)
def _build(n_tok: int, seq: int):
    tiles_per_w = n_tok // (NW * TB)
    n_blocks = seq * tiles_per_w
    assert n_blocks % 2 == 0 and tiles_per_w * NW * TB == n_tok
    mesh = plsc.VectorSubcoreMesh(
        core_axis_name="c", subcore_axis_name="s",
        num_cores=NC, num_subcores=NS,
    )

    @functools.partial(
        pl.kernel,
        out_type=jax.ShapeDtypeStruct((seq, D // 8, n_tok // TB, 8 * TB),
                                      jnp.float32),
        mesh=mesh,
        compiler_params=pltpu.CompilerParams(
            use_tc_tiling_on_sc=False, needs_layout_passes=False,
            disable_bounds_checks=True),
        scratch_types=[
            pltpu.VMEM((seq, tiles_per_w * TB), jnp.int32),
            pltpu.VMEM((TB, D), jnp.float32),
            pltpu.VMEM((TB, D), jnp.float32),
            pltpu.VMEM((D // 8, 8 * TB), jnp.float32),
            pltpu.VMEM((D // 8, 8 * TB), jnp.float32),
            pltpu.SemaphoreType.DMA,
            pltpu.SemaphoreType.DMA,
            pltpu.SemaphoreType.DMA,
            pltpu.SemaphoreType.DMA,
        ],
    )
    def body(idx_hbm, table_hbm, out_hbm, idx_v, rows_a, rows_b, t_a, t_b,
             gsem_a, gsem_b, osem_a, osem_b):
        wid = lax.axis_index("s") * NC + lax.axis_index("c")
        b0 = pl.multiple_of(wid * (tiles_per_w * TB), 8)
        lanes = lax.broadcasted_iota(jnp.int32, (16,), 0)
        rot = [(lanes + k) & 15 for k in range(16)]
        rot_hi = [lax.shift_right_logical(r, 3) for r in rot]
        rot_lo128 = [lax.shift_left(r & 7, 7) for r in rot]

        pltpu.sync_copy(idx_hbm.at[:, pl.ds(b0, tiles_per_w * TB)], idx_v)

        def fire_gather(n, buf, sem):
            j = n // tiles_per_w
            tbl = n % tiles_per_w
            pltpu.async_copy(
                table_hbm.at[idx_v.at[j, pl.ds(tbl * TB, TB)]], buf, sem)

        def wait_gather(buf, sem):
            pltpu.make_async_copy(table_hbm.at[pl.ds(0, TB)], buf, sem).wait()

        def transpose(buf, t_v):
            @pl.loop(0, TB // 16, unroll=2)
            def _(bi):
                bmv = bi * 16 + lanes
                for cb in range(D // 16):
                    vals = [plsc.load_gather(buf, [bmv, cb * 16 + rot[k]])
                            for k in range(16)]
                    for k in range(16):
                        plsc.store_scatter(
                            t_v, [cb * 2 + rot_hi[k], rot_lo128[k] + bmv],
                            vals[k])

        def fire_out(n, t_v, sem):
            j = n // tiles_per_w
            tb = wid * tiles_per_w + (n % tiles_per_w)
            pltpu.async_copy(t_v, out_hbm.at[j, :, tb], sem)

        def wait_out(t_v, sem):
            pltpu.make_async_copy(t_v, out_hbm.at[0, :, 0], sem).wait()

        fire_gather(0, rows_a, gsem_a)
        fire_out(0, t_a, osem_a)
        fire_out(1, t_b, osem_b)

        @pl.loop(0, n_blocks, step=2)
        def _(g0):
            fire_gather(g0 + 1, rows_b, gsem_b)
            wait_gather(rows_a, gsem_a)
            wait_out(t_a, osem_a)
            transpose(rows_a, t_a)
            fire_out(g0, t_a, osem_a)

            @pl.when(g0 + 2 < n_blocks)
            def _():
                fire_gather(g0 + 2, rows_a, gsem_a)

            wait_gather(rows_b, gsem_b)
            wait_out(t_b, osem_b)
            transpose(rows_b, t_b)
            fire_out(g0 + 1, t_b, osem_b)

        wait_out(t_a, osem_a)
        wait_out(t_b, osem_b)

    return body


def kernel(token_ids, weights):
    n_tok, seq = token_ids.shape
    out4 = _build(n_tok, seq)(token_ids.T.astype(jnp.int32), weights)
    out5 = out4.reshape(seq, D // 8, n_tok // TB, 8, TB)
    return out5.transpose(2, 4, 0, 1, 3).reshape(n_tok, seq, D)


# speedup vs baseline: 1.1324x; 1.0060x over previous
import functools

import numpy as np

import jax
import jax.numpy as jnp
from jax import lax
from jax.experimental import pallas as pl
from jax.experimental.pallas import tpu as pltpu
from jax.experimental.pallas import tpu_sc as plsc

NC = 2
NS = 16
NW = NC * NS
D = 64
TB = 128


@functools.lru_cache(maxsize=None)
def _build(n_tok: int, seq: int):
    tiles_per_w = n_tok // (NW * TB)
    n_blocks = seq * tiles_per_w
    assert n_blocks % 2 == 0 and tiles_per_w * NW * TB == n_tok
    mesh = plsc.VectorSubcoreMesh(
        core_axis_name="c", subcore_axis_name="s",
        num_cores=NC, num_subcores=NS,
    )

    @functools.partial(
        pl.kernel,
        out_type=jax.ShapeDtypeStruct((seq, D // 8, n_tok // TB, 8 * TB),
                                      jnp.float32),
        mesh=mesh,
        compiler_params=pltpu.CompilerParams(
            use_tc_tiling_on_sc=False, needs_layout_passes=False,
            disable_bounds_checks=True),
        scratch_types=[
            pltpu.VMEM((seq, tiles_per_w * TB), jnp.int32),
            pltpu.VMEM((TB, D), jnp.float32),
            pltpu.VMEM((TB, D), jnp.float32),
            pltpu.VMEM((TB, D), jnp.float32),
            pltpu.VMEM((TB, D), jnp.float32),
            pltpu.VMEM((D // 8, 8 * TB), jnp.float32),
            pltpu.VMEM((D // 8, 8 * TB), jnp.float32),
            pltpu.SemaphoreType.DMA,
            pltpu.SemaphoreType.DMA,
            pltpu.SemaphoreType.DMA,
            pltpu.SemaphoreType.DMA,
            pltpu.SemaphoreType.DMA,
            pltpu.SemaphoreType.DMA,
        ],
    )
    def body(idx_hbm, table_hbm, out_hbm, idx_v, rows_a, rows_b, rows_c,
             rows_d, t_a, t_b, gsem_a, gsem_b, gsem_c, gsem_d,
             osem_a, osem_b):
        wid = lax.axis_index("s") * NC + lax.axis_index("c")
        b0 = pl.multiple_of(wid * (tiles_per_w * TB), 8)
        lanes = lax.broadcasted_iota(jnp.int32, (16,), 0)
        rot = [(lanes + k) & 15 for k in range(16)]
        rot_hi = [lax.shift_right_logical(r, 3) for r in rot]
        rot_lo128 = [lax.shift_left(r & 7, 7) for r in rot]

        pltpu.sync_copy(idx_hbm.at[:, pl.ds(b0, tiles_per_w * TB)], idx_v)

        def fire_gather(n, buf, sem):
            j = n // tiles_per_w
            tbl = n % tiles_per_w
            pltpu.async_copy(
                table_hbm.at[idx_v.at[j, pl.ds(tbl * TB, TB)]], buf, sem)

        def wait_gather(buf, sem):
            pltpu.make_async_copy(table_hbm.at[pl.ds(0, TB)], buf, sem).wait()

        def transpose(buf, t_v):
            @pl.loop(0, TB // 16, unroll=2)
            def _(bi):
                bmv = bi * 16 + lanes
                for cb in range(D // 16):
                    vals = [plsc.load_gather(buf, [bmv, cb * 16 + rot[k]])
                            for k in range(16)]
                    for k in range(16):
                        plsc.store_scatter(
                            t_v, [cb * 2 + rot_hi[k], rot_lo128[k] + bmv],
                            vals[k])

        def fire_out(n, t_v, sem):
            j = n // tiles_per_w
            tb = wid * tiles_per_w + (n % tiles_per_w)
            pltpu.async_copy(t_v, out_hbm.at[j, :, tb], sem)

        def wait_out(t_v, sem):
            pltpu.make_async_copy(t_v, out_hbm.at[0, :, 0], sem).wait()

        assert n_blocks % 4 == 0
        fire_gather(0, rows_a, gsem_a)
        fire_gather(1, rows_b, gsem_b)
        fire_gather(2, rows_c, gsem_c)
        fire_out(0, t_a, osem_a)
        fire_out(1, t_b, osem_b)

        ring = [(rows_a, gsem_a), (rows_b, gsem_b),
                (rows_c, gsem_c), (rows_d, gsem_d)]
        touts = [(t_a, osem_a), (t_b, osem_b)]

        @pl.loop(0, n_blocks, step=4)
        def _(g0):
            pltpu.async_copy(
                table_hbm.at[idx_v.at[(g0 + 3) // tiles_per_w,
                                      pl.ds(((g0 + 3) % tiles_per_w) * TB,
                                            TB)]],
                rows_d, gsem_d)
            for s in range(4):
                buf, gsem = ring[s]
                t_v, osem = touts[s % 2]
                wait_gather(buf, gsem)
                wait_out(t_v, osem)
                transpose(buf, t_v)
                fire_out(g0 + s, t_v, osem)

                @pl.when(g0 + s + 4 < n_blocks)
                def _():
                    fire_gather(g0 + s + 4, buf, gsem)

        wait_out(t_a, osem_a)
        wait_out(t_b, osem_b)

    return body


def kernel(token_ids, weights):
    n_tok, seq = token_ids.shape
    out4 = _build(n_tok, seq)(token_ids.T.astype(jnp.int32), weights)
    out5 = out4.reshape(seq, D // 8, n_tok // TB, 8, TB)
    return out5.transpose(2, 4, 0, 1, 3).reshape(n_tok, seq, D)
